# baseline (speedup 1.0000x reference)
import jax
import jax.numpy as jnp
from jax.experimental import pallas as pl

SCALE_XY = 10.0
SCALE_WH = 5.0
ROWS = 1


def _smooth_l1(x):
    ax = jnp.abs(x)
    return jnp.where(ax < 1.0, 0.5 * x * x, ax - 0.5)


def _row_kernel(pred_box_ref, pred_lbl_ref, grd_box_ref, grd_lbl_ref,
                det_box_ref, con_ref, sl1_ref):
    logits = pred_lbl_ref[...]
    labels = grd_lbl_ref[...]
    R, C, P = logits.shape

    m = jnp.max(logits, axis=1, keepdims=True)
    s = jnp.sum(jnp.exp(logits - m), axis=1, keepdims=True)
    lse = jnp.log(s) + m
    cls = jax.lax.broadcasted_iota(jnp.int32, (R, C, P), 1)
    tl = jnp.sum(jnp.where(cls == labels, logits, 0.0), axis=1, keepdims=True)
    con_ref[...] = lse - tl

    maskf = (labels > 0).astype(jnp.float32)

    pb = pred_box_ref[...]
    gb = grd_box_ref[...]
    db = det_box_ref[...]
    gxy = SCALE_XY * (gb[:, :2] - db[:, :2]) / db[:, 2:]
    gwh = SCALE_WH * jnp.log(gb[:, 2:] / db[:, 2:])
    l = jnp.sum(_smooth_l1(pb[:, :2] - gxy), axis=1, keepdims=True)
    l = l + jnp.sum(_smooth_l1(pb[:, 2:] - gwh), axis=1, keepdims=True)
    lrow = jnp.sum(l * maskf, axis=2, keepdims=True)
    sl1_ref[...] = jnp.broadcast_to(lrow, (R, 1, 128))


def _select_kernel(con_ref, lbl_ref, sl1_ref, out_ref):
    con = con_ref[:, 0, :]
    labels = lbl_ref[:, 0, :]
    N, P = con.shape

    maskf = (labels > 0).astype(jnp.float32)
    posn = jnp.sum(maskf, axis=1, keepdims=True)
    kf = jnp.minimum(3.0 * posn, float(P))

    con_neg = jnp.where(labels > 0, 0.0, con)
    vb = jax.lax.bitcast_convert_type(con_neg, jnp.int32)

    def cnt_ge(cand):
        return jnp.sum((vb >= cand).astype(jnp.float32), axis=1, keepdims=True)

    def radix_body2(i, prefix):
        sh = jnp.int32(28) - 2 * i
        c1 = prefix | (jnp.int32(1) << sh)
        c2 = prefix | (jnp.int32(2) << sh)
        c3 = prefix | (jnp.int32(3) << sh)
        n1, n2, n3 = cnt_ge(c1), cnt_ge(c2), cnt_ge(c3)
        return jnp.where(n3 >= kf, c3,
               jnp.where(n2 >= kf, c2,
               jnp.where(n1 >= kf, c1, prefix)))

    def radix_body_top(prefix):
        cand = prefix | (jnp.int32(1) << 30)
        return jnp.where(cnt_ge(cand) >= kf, cand, prefix)

    t = radix_body_top(jnp.zeros((N, 1), jnp.int32))
    t = jax.lax.fori_loop(0, 15, radix_body2, t)

    gt = vb > t
    cnt_gt = jnp.sum(gt.astype(jnp.float32), axis=1, keepdims=True)
    slots = kf - cnt_gt
    eq = vb == t
    idx = jax.lax.broadcasted_iota(jnp.int32, (N, P), 1)

    def cnt_lt(cand):
        return jnp.sum((eq & (idx < cand)).astype(jnp.float32), axis=1,
                       keepdims=True)

    def tie_body2(i, u):
        sh = jnp.int32(12) - 2 * i
        c1 = u | (jnp.int32(1) << sh)
        c2 = u | (jnp.int32(2) << sh)
        c3 = u | (jnp.int32(3) << sh)
        f1, f2, f3 = cnt_lt(c1), cnt_lt(c2), cnt_lt(c3)
        return jnp.where(f3 < slots, c3,
               jnp.where(f2 < slots, c2,
               jnp.where(f1 < slots, c1, u)))

    u = jax.lax.fori_loop(0, 7, tie_body2, jnp.zeros((N, 1), jnp.int32))
    sel_eq = eq & (idx <= u) & (slots >= 0.5)

    negf = jnp.logical_or(gt, sel_eq).astype(jnp.float32)
    closs = jnp.sum(con * (maskf + negf), axis=1, keepdims=True)

    total = sl1_ref[:, 0, 0:1] + closs
    num_mask = (posn > 0).astype(jnp.float32)
    posc = jnp.maximum(posn, 1e-6)
    out_ref[:, :] = jnp.sum(total * num_mask / posc, keepdims=True) / N


def kernel(pred_box, pred_lbl, grd_box, grd_lbl, det_box):
    N, C, P = pred_lbl.shape
    lbl3 = grd_lbl.reshape(N, 1, P)
    R = ROWS

    con, sl1 = pl.pallas_call(
        _row_kernel,
        grid=(N // R,),
        in_specs=[
            pl.BlockSpec((R, 4, P), lambda i: (i, 0, 0)),
            pl.BlockSpec((R, C, P), lambda i: (i, 0, 0)),
            pl.BlockSpec((R, 4, P), lambda i: (i, 0, 0)),
            pl.BlockSpec((R, 1, P), lambda i: (i, 0, 0)),
            pl.BlockSpec((1, 4, P), lambda i: (0, 0, 0)),
        ],
        out_specs=[
            pl.BlockSpec((R, 1, P), lambda i: (i, 0, 0)),
            pl.BlockSpec((R, 1, 128), lambda i: (i, 0, 0)),
        ],
        out_shape=[
            jax.ShapeDtypeStruct((N, 1, P), jnp.float32),
            jax.ShapeDtypeStruct((N, 1, 128), jnp.float32),
        ],
    )(pred_box, pred_lbl, grd_box, lbl3, det_box)

    out = pl.pallas_call(
        _select_kernel,
        grid=(1,),
        in_specs=[
            pl.BlockSpec((N, 1, P), lambda i: (0, 0, 0)),
            pl.BlockSpec((N, 1, P), lambda i: (0, 0, 0)),
            pl.BlockSpec((N, 1, 128), lambda i: (0, 0, 0)),
        ],
        out_specs=pl.BlockSpec((1, 1), lambda i: (0, 0)),
        out_shape=jax.ShapeDtypeStruct((1, 1), jnp.float32),
    )(con, lbl3, sl1)
    return out[0, 0]


# speedup vs baseline: 1.2020x; 1.2020x over previous
import jax
import jax.numpy as jnp
from jax import lax
from jax.experimental import pallas as pl
from jax.experimental.pallas import tpu as pltpu

SCALE_XY = 10.0
SCALE_WH = 5.0
NSTREAM = 4
_CSPLIT = [0, 24, 48, 72]
_CMAIN = 72
_CTAIL = 9


def _smooth_l1(x):
    ax = jnp.abs(x)
    return jnp.where(ax < 1.0, 0.5 * x * x, ax - 0.5)


def _row_kernel(pred_box_ref, lbl_hbm, grd_box_ref, grd_lbl_ref,
                det_box_ref, con_ref, sl1_ref, lbuf, fbuf, sems):
    i = pl.program_id(0)
    nrow = pl.num_programs(0)

    def copies(row, slot):
        out = []
        for k in range(NSTREAM - 1):
            lo = _CSPLIT[k]
            out.append(pltpu.make_async_copy(
                lbl_hbm.at[row, pl.ds(lo, 24), :],
                lbuf.at[slot, pl.ds(lo, 24), :],
                sems.at[slot, k]))
        out.append(pltpu.make_async_copy(
            lbl_hbm.at[row, pl.ds(_CMAIN, _CTAIL), :],
            fbuf.at[slot],
            sems.at[slot, NSTREAM - 1]))
        return out

    @pl.when(i == 0)
    def _():
        for c in copies(0, 0):
            c.start()

    @pl.when(i + 1 < nrow)
    def _():
        for c in copies(i + 1, lax.rem(i + 1, 2)):
            c.start()

    slot = lax.rem(i, 2)
    for c in copies(i, slot):
        c.wait()

    la = lbuf[slot]
    lb = fbuf[slot]
    labels = grd_lbl_ref[0]
    P = la.shape[1]

    m = jnp.maximum(jnp.max(la, axis=0, keepdims=True),
                    jnp.max(lb, axis=0, keepdims=True))
    s = (jnp.sum(jnp.exp(la - m), axis=0, keepdims=True)
         + jnp.sum(jnp.exp(lb - m), axis=0, keepdims=True))
    lse = jnp.log(s) + m
    cls_a = jax.lax.broadcasted_iota(jnp.int32, (_CMAIN, P), 0)
    cls_b = jax.lax.broadcasted_iota(jnp.int32, (_CTAIL, P), 0) + _CMAIN
    tl = (jnp.sum(jnp.where(cls_a == labels, la, 0.0), axis=0, keepdims=True)
          + jnp.sum(jnp.where(cls_b == labels, lb, 0.0), axis=0, keepdims=True))
    con_ref[0] = lse - tl

    maskf = (labels > 0).astype(jnp.float32)

    pb = pred_box_ref[0]
    gb = grd_box_ref[0]
    db = det_box_ref[0]
    gxy = SCALE_XY * (gb[:2] - db[:2]) / db[2:]
    gwh = SCALE_WH * jnp.log(gb[2:] / db[2:])
    l = jnp.sum(_smooth_l1(pb[:2] - gxy), axis=0, keepdims=True)
    l = l + jnp.sum(_smooth_l1(pb[2:] - gwh), axis=0, keepdims=True)
    lrow = jnp.sum(l * maskf, keepdims=True)
    sl1_ref[0] = jnp.broadcast_to(lrow, (1, 128))


def _select_kernel(con_ref, lbl_ref, sl1_ref, out_ref):
    con = con_ref[:, 0, :]
    labels = lbl_ref[:, 0, :]
    N, P = con.shape

    maskf = (labels > 0).astype(jnp.float32)
    posn = jnp.sum(maskf, axis=1, keepdims=True)
    kf = jnp.minimum(3.0 * posn, float(P))

    con_neg = jnp.where(labels > 0, 0.0, con)
    vb = jax.lax.bitcast_convert_type(con_neg, jnp.int32)

    def radix_body(i, prefix):
        cand = prefix | (jnp.int32(1) << (jnp.int32(30) - i))
        cnt = jnp.sum((vb >= cand).astype(jnp.float32), axis=1, keepdims=True)
        return jnp.where(cnt >= kf, cand, prefix)

    t = jax.lax.fori_loop(0, 31, radix_body, jnp.zeros((N, 1), jnp.int32))

    gt = vb > t
    cnt_gt = jnp.sum(gt.astype(jnp.float32), axis=1, keepdims=True)
    slots = kf - cnt_gt
    eq = vb == t
    idx = jax.lax.broadcasted_iota(jnp.int32, (N, P), 1)

    def tie_body(i, u):
        cand = u | (jnp.int32(1) << (jnp.int32(13) - i))
        f = jnp.sum((eq & (idx < cand)).astype(jnp.float32), axis=1,
                    keepdims=True)
        return jnp.where(f < slots, cand, u)

    u = jax.lax.fori_loop(0, 14, tie_body, jnp.zeros((N, 1), jnp.int32))
    sel_eq = eq & (idx <= u) & (slots >= 0.5)

    negf = jnp.logical_or(gt, sel_eq).astype(jnp.float32)
    closs = jnp.sum(con * (maskf + negf), axis=1, keepdims=True)

    total = sl1_ref[:, 0, 0:1] + closs
    num_mask = (posn > 0).astype(jnp.float32)
    posc = jnp.maximum(posn, 1e-6)
    out_ref[:, :] = jnp.sum(total * num_mask / posc, keepdims=True) / N


def kernel(pred_box, pred_lbl, grd_box, grd_lbl, det_box):
    N, C, P = pred_lbl.shape
    lbl3 = grd_lbl.reshape(N, 1, P)

    con, sl1 = pl.pallas_call(
        _row_kernel,
        grid=(N,),
        in_specs=[
            pl.BlockSpec((1, 4, P), lambda i: (i, 0, 0)),
            pl.BlockSpec(memory_space=pl.ANY),
            pl.BlockSpec((1, 4, P), lambda i: (i, 0, 0)),
            pl.BlockSpec((1, 1, P), lambda i: (i, 0, 0)),
            pl.BlockSpec((1, 4, P), lambda i: (0, 0, 0)),
        ],
        out_specs=[
            pl.BlockSpec((1, 1, P), lambda i: (i, 0, 0)),
            pl.BlockSpec((1, 1, 128), lambda i: (i, 0, 0)),
        ],
        out_shape=[
            jax.ShapeDtypeStruct((N, 1, P), jnp.float32),
            jax.ShapeDtypeStruct((N, 1, 128), jnp.float32),
        ],
        scratch_shapes=[
            pltpu.VMEM((2, _CMAIN, P), jnp.float32),
            pltpu.VMEM((2, _CTAIL, P), jnp.float32),
            pltpu.SemaphoreType.DMA((2, NSTREAM)),
        ],
    )(pred_box, pred_lbl, grd_box, lbl3, det_box)

    out = pl.pallas_call(
        _select_kernel,
        grid=(1,),
        in_specs=[
            pl.BlockSpec((N, 1, P), lambda i: (0, 0, 0)),
            pl.BlockSpec((N, 1, P), lambda i: (0, 0, 0)),
            pl.BlockSpec((N, 1, 128), lambda i: (0, 0, 0)),
        ],
        out_specs=pl.BlockSpec((1, 1), lambda i: (0, 0)),
        out_shape=jax.ShapeDtypeStruct((1, 1), jnp.float32),
    )(con, lbl3, sl1)
    return out[0, 0]


# speedup vs baseline: 1.2174x; 1.0128x over previous
import jax
import jax.numpy as jnp
from jax.experimental import pallas as pl

SCALE_XY = 10.0
SCALE_WH = 5.0


def _smooth_l1(x):
    ax = jnp.abs(x)
    return jnp.where(ax < 1.0, 0.5 * x * x, ax - 0.5)


def _row_kernel(pred_box_ref, pred_lbl_ref, grd_box_ref, grd_lbl_ref,
                det_box_ref, con_ref, sl1_ref):
    logits = pred_lbl_ref[0]
    labels = grd_lbl_ref[0]
    C, P = logits.shape

    m = jnp.max(logits, axis=0, keepdims=True)
    s = jnp.sum(jnp.exp(logits - m), axis=0, keepdims=True)
    lse = jnp.log(s) + m
    cls = jax.lax.broadcasted_iota(jnp.int32, (C, P), 0)
    tl = jnp.sum(jnp.where(cls == labels, logits, 0.0), axis=0, keepdims=True)
    con_ref[0] = lse - tl

    maskf = (labels > 0).astype(jnp.float32)

    pb = pred_box_ref[0]
    gb = grd_box_ref[0]
    db = det_box_ref[0]
    gxy = SCALE_XY * (gb[:2] - db[:2]) / db[2:]
    gwh = SCALE_WH * jnp.log(gb[2:] / db[2:])
    l = jnp.sum(_smooth_l1(pb[:2] - gxy), axis=0, keepdims=True)
    l = l + jnp.sum(_smooth_l1(pb[2:] - gwh), axis=0, keepdims=True)
    lrow = jnp.sum(l * maskf, keepdims=True)
    sl1_ref[0] = jnp.broadcast_to(lrow, (1, 128))


def _select_kernel(con_ref, lbl_ref, sl1_ref, out_ref):
    con = con_ref[:, 0, :]
    labels = lbl_ref[:, 0, :]
    N, P = con.shape

    maskf = (labels > 0).astype(jnp.float32)
    posn = jnp.sum(maskf, axis=1, keepdims=True)
    kf = jnp.minimum(3.0 * posn, float(P))

    con_neg = jnp.where(labels > 0, 0.0, con)
    vb = jax.lax.bitcast_convert_type(con_neg, jnp.int32)

    def radix_body(i, prefix):
        cand = prefix | (jnp.int32(1) << (jnp.int32(30) - i))
        cnt = jnp.sum((vb >= cand).astype(jnp.float32), axis=1, keepdims=True)
        return jnp.where(cnt >= kf, cand, prefix)

    t = jax.lax.fori_loop(0, 31, radix_body, jnp.zeros((N, 1), jnp.int32))

    gt = vb > t
    cnt_gt = jnp.sum(gt.astype(jnp.float32), axis=1, keepdims=True)
    slots = kf - cnt_gt
    eq = vb == t
    idx = jax.lax.broadcasted_iota(jnp.int32, (N, P), 1)

    def tie_body(i, u):
        cand = u | (jnp.int32(1) << (jnp.int32(13) - i))
        f = jnp.sum((eq & (idx < cand)).astype(jnp.float32), axis=1,
                    keepdims=True)
        return jnp.where(f < slots, cand, u)

    u = jax.lax.fori_loop(0, 14, tie_body, jnp.zeros((N, 1), jnp.int32))
    sel_eq = eq & (idx <= u) & (slots >= 0.5)

    negf = jnp.logical_or(gt, sel_eq).astype(jnp.float32)
    closs = jnp.sum(con * (maskf + negf), axis=1, keepdims=True)

    total = sl1_ref[:, 0, 0:1] + closs
    num_mask = (posn > 0).astype(jnp.float32)
    posc = jnp.maximum(posn, 1e-6)
    out_ref[:, :] = jnp.sum(total * num_mask / posc, keepdims=True) / N


def kernel(pred_box, pred_lbl, grd_box, grd_lbl, det_box):
    N, C, P = pred_lbl.shape
    lbl3 = grd_lbl.reshape(N, 1, P)

    con, sl1 = pl.pallas_call(
        _row_kernel,
        grid=(N,),
        in_specs=[
            pl.BlockSpec((1, 4, P), lambda i: (i, 0, 0)),
            pl.BlockSpec((1, C, P), lambda i: (i, 0, 0)),
            pl.BlockSpec((1, 4, P), lambda i: (i, 0, 0)),
            pl.BlockSpec((1, 1, P), lambda i: (i, 0, 0)),
            pl.BlockSpec((1, 4, P), lambda i: (0, 0, 0)),
        ],
        out_specs=[
            pl.BlockSpec((1, 1, P), lambda i: (i, 0, 0)),
            pl.BlockSpec((1, 1, 128), lambda i: (i, 0, 0)),
        ],
        out_shape=[
            jax.ShapeDtypeStruct((N, 1, P), jnp.float32),
            jax.ShapeDtypeStruct((N, 1, 128), jnp.float32),
        ],
    )(pred_box, pred_lbl, grd_box, lbl3, det_box)

    out = pl.pallas_call(
        _select_kernel,
        grid=(1,),
        in_specs=[
            pl.BlockSpec((N, 1, P), lambda i: (0, 0, 0)),
            pl.BlockSpec((N, 1, P), lambda i: (0, 0, 0)),
            pl.BlockSpec((N, 1, 128), lambda i: (0, 0, 0)),
        ],
        out_specs=pl.BlockSpec((1, 1), lambda i: (0, 0)),
        out_shape=jax.ShapeDtypeStruct((1, 1), jnp.float32),
    )(con, lbl3, sl1)
    return out[0, 0]


# speedup vs baseline: 1.3024x; 1.0699x over previous
import jax
import jax.numpy as jnp
from jax.experimental import pallas as pl

SCALE_XY = 10.0
SCALE_WH = 5.0


def _smooth_l1(x):
    ax = jnp.abs(x)
    return jnp.where(ax < 1.0, 0.5 * x * x, ax - 0.5)


def _row_kernel(pred_box_ref, pred_lbl_ref, grd_box_ref, grd_lbl_ref,
                det_box_ref, con_ref, sl1_ref):
    logits = pred_lbl_ref[0]
    labels = grd_lbl_ref[0]
    C, P = logits.shape

    m = jnp.max(logits, axis=0, keepdims=True)
    s = jnp.sum(jnp.exp(logits - m), axis=0, keepdims=True)
    lse = jnp.log(s) + m
    cls = jax.lax.broadcasted_iota(jnp.int32, (C, P), 0)
    tl = jnp.sum(jnp.where(cls == labels, logits, 0.0), axis=0, keepdims=True)
    con_ref[0] = lse - tl

    maskf = (labels > 0).astype(jnp.float32)

    pb = pred_box_ref[0]
    gb = grd_box_ref[0]
    db = det_box_ref[0]
    gxy = SCALE_XY * (gb[:2] - db[:2]) / db[2:]
    gwh = SCALE_WH * jnp.log(gb[2:] / db[2:])
    l = jnp.sum(_smooth_l1(pb[:2] - gxy), axis=0, keepdims=True)
    l = l + jnp.sum(_smooth_l1(pb[2:] - gwh), axis=0, keepdims=True)
    lrow = jnp.sum(l * maskf, keepdims=True)
    sl1_ref[0] = jnp.broadcast_to(lrow, (1, 128))


def _select_kernel(con_ref, lbl_ref, sl1_ref, out_ref):
    con = con_ref[:, 0, :]
    labels = lbl_ref[:, 0, :]
    N, P = con.shape

    maskf = (labels > 0).astype(jnp.float32)
    posn = jnp.sum(maskf, axis=1, keepdims=True)
    kf = jnp.minimum(3.0 * posn, float(P))

    con_neg = jnp.where(labels > 0, 0.0, con)
    vb = jax.lax.bitcast_convert_type(con_neg, jnp.int32)

    def _rowsum(x):
        a = jnp.sum(x[:, :2304], axis=1, keepdims=True)
        b = jnp.sum(x[:, 2304:4608], axis=1, keepdims=True)
        c = jnp.sum(x[:, 4608:6912], axis=1, keepdims=True)
        d = jnp.sum(x[:, 6912:], axis=1, keepdims=True)
        return (a + b) + (c + d)

    def radix_body(i, prefix):
        cand = prefix | (jnp.int32(1) << (jnp.int32(30) - i))
        cnt = _rowsum((vb >= cand).astype(jnp.float32))
        return jnp.where(cnt >= kf, cand, prefix)

    t = jax.lax.fori_loop(0, 31, radix_body, jnp.zeros((N, 1), jnp.int32))

    gt = vb > t
    cnt_gt = jnp.sum(gt.astype(jnp.float32), axis=1, keepdims=True)
    slots = kf - cnt_gt
    eq = vb == t
    idx = jax.lax.broadcasted_iota(jnp.int32, (N, P), 1)

    def tie_body(i, u):
        cand = u | (jnp.int32(1) << (jnp.int32(13) - i))
        f = _rowsum((eq & (idx < cand)).astype(jnp.float32))
        return jnp.where(f < slots, cand, u)

    u = jax.lax.fori_loop(0, 14, tie_body, jnp.zeros((N, 1), jnp.int32))
    sel_eq = eq & (idx <= u) & (slots >= 0.5)

    negf = jnp.logical_or(gt, sel_eq).astype(jnp.float32)
    closs = jnp.sum(con * (maskf + negf), axis=1, keepdims=True)

    total = sl1_ref[:, 0, 0:1] + closs
    num_mask = (posn > 0).astype(jnp.float32)
    posc = jnp.maximum(posn, 1e-6)
    out_ref[:, :] = jnp.sum(total * num_mask / posc, keepdims=True) / N


def kernel(pred_box, pred_lbl, grd_box, grd_lbl, det_box):
    N, C, P = pred_lbl.shape
    lbl3 = grd_lbl.reshape(N, 1, P)

    con, sl1 = pl.pallas_call(
        _row_kernel,
        grid=(N,),
        in_specs=[
            pl.BlockSpec((1, 4, P), lambda i: (i, 0, 0)),
            pl.BlockSpec((1, C, P), lambda i: (i, 0, 0)),
            pl.BlockSpec((1, 4, P), lambda i: (i, 0, 0)),
            pl.BlockSpec((1, 1, P), lambda i: (i, 0, 0)),
            pl.BlockSpec((1, 4, P), lambda i: (0, 0, 0)),
        ],
        out_specs=[
            pl.BlockSpec((1, 1, P), lambda i: (i, 0, 0)),
            pl.BlockSpec((1, 1, 128), lambda i: (i, 0, 0)),
        ],
        out_shape=[
            jax.ShapeDtypeStruct((N, 1, P), jnp.float32),
            jax.ShapeDtypeStruct((N, 1, 128), jnp.float32),
        ],
    )(pred_box, pred_lbl, grd_box, lbl3, det_box)

    out = pl.pallas_call(
        _select_kernel,
        grid=(1,),
        in_specs=[
            pl.BlockSpec((N, 1, P), lambda i: (0, 0, 0)),
            pl.BlockSpec((N, 1, P), lambda i: (0, 0, 0)),
            pl.BlockSpec((N, 1, 128), lambda i: (0, 0, 0)),
        ],
        out_specs=pl.BlockSpec((1, 1), lambda i: (0, 0)),
        out_shape=jax.ShapeDtypeStruct((1, 1), jnp.float32),
    )(con, lbl3, sl1)
    return out[0, 0]
